# baseline (speedup 1.0000x reference)
import functools

import jax
import jax.numpy as jnp
from jax import lax
from jax.experimental import pallas as pl
from jax.experimental.pallas import tpu as pltpu
from jax.experimental.pallas import tpu_sc as plsc

_BUCKETS = 100000
_D = 64
_DP = 128
_NC = 2
_NS = 16
_NW = _NC * _NS
_BB = 128


@functools.partial(jax.jit, static_argnames=("nb", "ns"))
def _sc_gather(ids_t, table_padded, nb, ns):
    n_blk = nb // _BB
    n_units = ns * n_blk
    u_per_w = n_units // _NW
    n_per_w = u_per_w * _BB
    mesh = plsc.VectorSubcoreMesh(core_axis_name="c", subcore_axis_name="s")

    @functools.partial(
        pl.kernel,
        out_type=jax.ShapeDtypeStruct((ns, _D, nb), jnp.float32),
        mesh=mesh,
        scratch_types=[
            pltpu.VMEM((n_per_w,), jnp.int32),
            pltpu.VMEM((_BB, _DP), jnp.float32),
            pltpu.VMEM((_BB, _DP), jnp.float32),
            pltpu.VMEM((_D, _BB), jnp.float32),
            pltpu.VMEM((_D, _BB), jnp.float32),
            pltpu.SemaphoreType.DMA,
            pltpu.SemaphoreType.DMA,
            pltpu.SemaphoreType.DMA,
            pltpu.SemaphoreType.DMA,
        ],
        compiler_params=pltpu.CompilerParams(
            use_tc_tiling_on_sc=True, needs_layout_passes=False),
    )
    def k(ids_hbm, table_hbm, out_hbm, idx_v, grows0, grows1, trows0, trows1,
          gsem0, gsem1, osem0, osem1):
        grows = (grows0, grows1)
        trows = (trows0, trows1)
        gsem = (gsem0, gsem1)
        osem = (osem0, osem1)

        wid = lax.axis_index("s") * _NC + lax.axis_index("c")
        base = wid * n_per_w
        u_base = wid * u_per_w

        pltpu.sync_copy(ids_hbm.at[pl.ds(base, n_per_w)], idx_v)

        @pl.loop(0, n_per_w // 16, step=8)
        def _mod(i):
            for j in range(8):
                sl = pl.ds((i + j) * 16, 16)
                x = idx_v[sl]
                for c in (8 * _BUCKETS, 4 * _BUCKETS, 2 * _BUCKETS, _BUCKETS):
                    x = jnp.where(x >= c, x - c, x)
                idx_v[sl] = x

        bvecs = [lax.iota(jnp.int32, 16) + (bb * 16) for bb in range(8)]

        def gather_desc(g, p):
            return pltpu.make_async_copy(
                table_hbm.at[idx_v.at[pl.ds(g * _BB, _BB)]],
                grows[p],
                gsem[p],
            )

        def out_desc(g, p):
            u = u_base + g
            s = u // n_blk
            b0 = (u % n_blk) * _BB
            return pltpu.make_async_copy(
                trows[p],
                out_hbm.at[s, :, pl.ds(b0, _BB)],
                osem[p],
            )

        def transpose(p):
            g_ref = grows[p]
            t_ref = trows[p]

            @pl.loop(0, _D, step=4)
            def _t(d0):
                for dd in range(4):
                    d = d0 + dd
                    dvec = jnp.broadcast_to(d, (16,)).astype(jnp.int32)
                    for bb in range(8):
                        seg = plsc.load_gather(g_ref, [bvecs[bb], dvec])
                        t_ref[d, pl.ds(bb * 16, 16)] = seg

        gather_desc(0, 0).start()

        @pl.loop(0, u_per_w, step=2)
        def _main(g0):
            for p in range(2):
                g = g0 + p

                @pl.when(g + 1 < u_per_w)
                def _fire_next():
                    gather_desc(g + 1, 1 - p).start()

                gather_desc(g, p).wait()

                @pl.when(g >= 2)
                def _wait_prev_out():
                    out_desc(g - 2, p).wait()

                transpose(p)
                out_desc(g, p).start()

        out_desc(u_per_w - 2, 0).wait()
        out_desc(u_per_w - 1, 1).wait()

    return k(ids_t, table_padded)


def kernel(token_ids, bucket_embeddings):
    nb, ns = token_ids.shape
    ids_t = token_ids.T.reshape(nb * ns).astype(jnp.int32)
    table_padded = jnp.pad(bucket_embeddings, ((0, 0), (0, _DP - _D)))
    out_t = _sc_gather(ids_t, table_padded, nb, ns)
    return jnp.transpose(out_t, (2, 0, 1))


# speedup vs baseline: 1.2168x; 1.2168x over previous
import functools

import jax
import jax.numpy as jnp
from jax import lax
from jax.experimental import pallas as pl
from jax.experimental.pallas import tpu as pltpu
from jax.experimental.pallas import tpu_sc as plsc

_BUCKETS = 100000
_D = 64
_DP = 128
_NC = 2
_NS = 16
_NW = _NC * _NS
_BB = 128
_TP = 129


@functools.partial(jax.jit, static_argnames=("nb", "ns"))
def _sc_gather(ids_t, table_padded, nb, ns):
    n_blk = nb // _BB
    n_units = ns * n_blk
    u_per_w = n_units // _NW
    n_per_w = u_per_w * _BB
    mesh = plsc.VectorSubcoreMesh(core_axis_name="c", subcore_axis_name="s")

    @functools.partial(
        pl.kernel,
        out_type=jax.ShapeDtypeStruct((ns, _D, nb), jnp.float32),
        mesh=mesh,
        scratch_types=[
            pltpu.VMEM((n_per_w,), jnp.int32),
            pltpu.VMEM((_BB, _DP), jnp.float32),
            pltpu.VMEM((_BB, _DP), jnp.float32),
            pltpu.VMEM((_D, _TP), jnp.float32),
            pltpu.VMEM((_D, _TP), jnp.float32),
            pltpu.SemaphoreType.DMA,
            pltpu.SemaphoreType.DMA,
            pltpu.SemaphoreType.DMA,
            pltpu.SemaphoreType.DMA,
        ],
        compiler_params=pltpu.CompilerParams(
            use_tc_tiling_on_sc=True, needs_layout_passes=False),
    )
    def k(ids_hbm, table_hbm, out_hbm, idx_v, grows0, grows1, trows0, trows1,
          gsem0, gsem1, osem0, osem1):
        grows = (grows0, grows1)
        trows = (trows0, trows1)
        gsem = (gsem0, gsem1)
        osem = (osem0, osem1)

        wid = lax.axis_index("s") * _NC + lax.axis_index("c")
        base = wid * n_per_w
        u_base = wid * u_per_w

        pltpu.sync_copy(ids_hbm.at[pl.ds(base, n_per_w)], idx_v)

        @pl.loop(0, n_per_w // 16, step=8)
        def _mod(i):
            for j in range(8):
                sl = pl.ds((i + j) * 16, 16)
                x = idx_v[sl]
                for c in (8 * _BUCKETS, 4 * _BUCKETS, 2 * _BUCKETS, _BUCKETS):
                    x = jnp.where(x >= c, x - c, x)
                idx_v[sl] = x

        dvecs = [lax.iota(jnp.int32, 16) + (q * 16) for q in range(_D // 16)]

        def gather_desc(g, p):
            return pltpu.make_async_copy(
                table_hbm.at[idx_v.at[pl.ds(g * _BB, _BB)]],
                grows[p],
                gsem[p],
            )

        def out_desc(g, p):
            u = u_base + g
            s = u // n_blk
            b0 = (u % n_blk) * _BB
            return pltpu.make_async_copy(
                trows[p].at[:, pl.ds(0, _BB)],
                out_hbm.at[s, :, pl.ds(b0, _BB)],
                osem[p],
            )

        def transpose(p):
            g_ref = grows[p]
            t_ref = trows[p]

            @pl.loop(0, _BB, step=2)
            def _t(b0):
                for db in range(2):
                    b = b0 + db
                    bvec = jnp.broadcast_to(b, (16,)).astype(jnp.int32)
                    for q in range(_D // 16):
                        seg = g_ref[b, pl.ds(q * 16, 16)]
                        plsc.store_scatter(t_ref, [dvecs[q], bvec], seg)

        gather_desc(0, 0).start()

        @pl.loop(0, u_per_w, step=2)
        def _main(g0):
            for p in range(2):
                g = g0 + p

                @pl.when(g + 1 < u_per_w)
                def _fire_next():
                    gather_desc(g + 1, 1 - p).start()

                gather_desc(g, p).wait()

                @pl.when(g >= 2)
                def _wait_prev_out():
                    out_desc(g - 2, p).wait()

                transpose(p)
                out_desc(g, p).start()

        out_desc(u_per_w - 2, 0).wait()
        out_desc(u_per_w - 1, 1).wait()

    return k(ids_t, table_padded)


def kernel(token_ids, bucket_embeddings):
    nb, ns = token_ids.shape
    ids_t = token_ids.T.reshape(nb * ns).astype(jnp.int32)
    table_padded = jnp.pad(bucket_embeddings, ((0, 0), (0, _DP - _D)))
    out_t = _sc_gather(ids_t, table_padded, nb, ns)
    return jnp.transpose(out_t, (2, 0, 1))
